# baseline (speedup 1.0000x reference)
import functools

import jax
import jax.numpy as jnp
from jax import lax
from jax.experimental import pallas as pl
from jax.experimental.pallas import tpu as pltpu
from jax.experimental.pallas import tpu_sc as plsc

_VOCAB = 1000000
_D = 64
_B = 4096
_T = 200
_N = _B * _T

_NC = 2
_NS = 16
_NW = _NC * _NS
_PER_W = _N // _NW
_CHUNK = 128
_NCHUNK = _PER_W // _CHUNK

_mesh = plsc.VectorSubcoreMesh(core_axis_name="c", subcore_axis_name="s")


@functools.partial(
    pl.kernel,
    mesh=_mesh,
    out_type=jax.ShapeDtypeStruct((_N, _D), jnp.float32),
    compiler_params=pltpu.CompilerParams(use_tc_tiling_on_sc=False),
    scratch_types=[
        pltpu.VMEM((_CHUNK,), jnp.int32),
        pltpu.VMEM((_CHUNK, _D), jnp.float32),
        pltpu.SemaphoreType.DMA,
    ],
)
def _gather_kernel(idx_hbm, table_hbm, out_hbm, idx_v, rows_v, sem):
    wid = lax.axis_index("s") * _NC + lax.axis_index("c")
    base = wid * _PER_W

    def body(j, carry):
        off = base + j * _CHUNK
        pltpu.sync_copy(idx_hbm.at[pl.ds(off, _CHUNK)], idx_v)
        pltpu.async_copy(table_hbm.at[idx_v], rows_v, sem).wait()
        pltpu.sync_copy(rows_v, out_hbm.at[pl.ds(off, _CHUNK)])
        return carry

    lax.fori_loop(0, _NCHUNK, body, 0)


def kernel(idx, table):
    flat_idx = jnp.asarray(idx, jnp.int32).reshape(_N)
    out = _gather_kernel(flat_idx, table)
    return out.reshape(_B, _T, _D)


# speedup vs baseline: 1.1937x; 1.1937x over previous
import functools

import jax
import jax.numpy as jnp
from jax import lax
from jax.experimental import pallas as pl
from jax.experimental.pallas import tpu as pltpu
from jax.experimental.pallas import tpu_sc as plsc

_VOCAB = 1000000
_D = 64
_B = 4096
_T = 200
_N = _B * _T

_NC = 2
_NS = 16
_NW = _NC * _NS
_PER_W = _N // _NW
_CHUNK = 512
_NCHUNK = _PER_W // _CHUNK

_mesh = plsc.VectorSubcoreMesh(core_axis_name="c", subcore_axis_name="s")


@functools.partial(
    pl.kernel,
    mesh=_mesh,
    out_type=jax.ShapeDtypeStruct((_N, _D), jnp.float32),
    compiler_params=pltpu.CompilerParams(use_tc_tiling_on_sc=False),
    scratch_types=[
        pltpu.VMEM((_PER_W,), jnp.int32),
        pltpu.VMEM((_CHUNK, _D), jnp.float32),
        pltpu.VMEM((_CHUNK, _D), jnp.float32),
        pltpu.SemaphoreType.DMA,
        pltpu.SemaphoreType.DMA,
        pltpu.SemaphoreType.DMA,
        pltpu.SemaphoreType.DMA,
    ],
)
def _gather_kernel(idx_hbm, table_hbm, out_hbm, idx_v, rb0, rb1,
                   g0, g1, w0, w1):
    wid = lax.axis_index("s") * _NC + lax.axis_index("c")
    base = wid * _PER_W

    pltpu.sync_copy(idx_hbm.at[pl.ds(base, _PER_W)], idx_v)

    def fire_gather(j, rbuf, gsem):
        pltpu.make_async_copy(
            table_hbm.at[idx_v.at[pl.ds(j * _CHUNK, _CHUNK)]], rbuf, gsem
        ).start()

    def wait_gather(rbuf, gsem):
        pltpu.make_async_copy(
            table_hbm.at[idx_v.at[pl.ds(0, _CHUNK)]], rbuf, gsem
        ).wait()

    def fire_write(j, rbuf, wsem):
        pltpu.make_async_copy(
            rbuf, out_hbm.at[pl.ds(base + j * _CHUNK, _CHUNK)], wsem
        ).start()

    def wait_write(j, rbuf, wsem):
        pltpu.make_async_copy(
            rbuf, out_hbm.at[pl.ds(base + j * _CHUNK, _CHUNK)], wsem
        ).wait()

    fire_gather(0, rb0, g0)
    fire_gather(1, rb1, g1)

    def body(i2, carry):
        for b, (rbuf, gsem, wsem) in enumerate(((rb0, g0, w0), (rb1, g1, w1))):
            j = i2 * 2 + b
            wait_gather(rbuf, gsem)
            fire_write(j, rbuf, wsem)
            wait_write(j, rbuf, wsem)

            @pl.when(j + 2 < _NCHUNK)
            def _():
                fire_gather(j + 2, rbuf, gsem)

        return carry

    lax.fori_loop(0, _NCHUNK // 2, body, 0)


def kernel(idx, table):
    flat_idx = jnp.asarray(idx, jnp.int32).reshape(_N)
    out = _gather_kernel(flat_idx, table)
    return out.reshape(_B, _T, _D)


# speedup vs baseline: 1.5054x; 1.2612x over previous
import functools

import jax
import jax.numpy as jnp
from jax import lax
from jax.experimental import pallas as pl
from jax.experimental.pallas import tpu as pltpu
from jax.experimental.pallas import tpu_sc as plsc

_VOCAB = 1000000
_D = 64
_B = 4096
_T = 200
_N = _B * _T

_NC = 2
_NS = 16
_NW = _NC * _NS
_PER_W = _N // _NW
_CHUNK = 512
_NCHUNK = _PER_W // _CHUNK

_RB = 4096
_HALF = _RB // 2
_TGRID = -(-_VOCAB // _RB)
_VP = _TGRID * _RB

_mesh = plsc.VectorSubcoreMesh(core_axis_name="c", subcore_axis_name="s")


def _transpose_body(t_ref, o_ref):
    x = t_ref[...]
    xt = jnp.transpose(x)
    lo = lax.slice(xt, (0, 0), (_HALF, _D))
    hi = lax.slice(xt, (_HALF, 0), (_RB, _D))
    o_ref[...] = jnp.concatenate([lo, hi], axis=1)


_transpose_tc = pl.pallas_call(
    _transpose_body,
    grid=(_TGRID,),
    in_specs=[pl.BlockSpec((_D, _RB), lambda i: (0, i))],
    out_specs=pl.BlockSpec((_HALF, 128), lambda i: (i, 0)),
    out_shape=jax.ShapeDtypeStruct((_VP // 2, 128), jnp.float32),
)


@functools.partial(
    pl.kernel,
    mesh=_mesh,
    out_type=jax.ShapeDtypeStruct((_N, _D), jnp.float32),
    compiler_params=pltpu.CompilerParams(use_tc_tiling_on_sc=False),
    scratch_types=[
        pltpu.VMEM((_PER_W,), jnp.int32),
        pltpu.VMEM((_CHUNK, _D), jnp.float32),
        pltpu.VMEM((_CHUNK, _D), jnp.float32),
        pltpu.SemaphoreType.DMA,
        pltpu.SemaphoreType.DMA,
        pltpu.SemaphoreType.DMA,
        pltpu.SemaphoreType.DMA,
    ],
)
def _gather_kernel(idx_hbm, table_hbm, out_hbm, idx_v, rb0, rb1,
                   g0, g1, w0, w1):
    wid = lax.axis_index("s") * _NC + lax.axis_index("c")
    base = wid * _PER_W

    pltpu.sync_copy(idx_hbm.at[pl.ds(base, _PER_W)], idx_v)

    def tbody(i, carry):
        v = idx_v[pl.ds(i * 16, 16)]
        blk = lax.shift_left(lax.shift_right_logical(v, 12), 12)
        k = blk + lax.shift_left(v & 2047, 1) + (lax.shift_right_logical(v, 11) & 1)
        idx_v[pl.ds(i * 16, 16)] = k
        return carry

    lax.fori_loop(0, _PER_W // 16, tbody, 0)

    def fire_gather(j, rbuf, gsem):
        pltpu.make_async_copy(
            table_hbm.at[idx_v.at[pl.ds(j * _CHUNK, _CHUNK)]], rbuf, gsem
        ).start()

    def wait_gather(rbuf, gsem):
        pltpu.make_async_copy(
            table_hbm.at[idx_v.at[pl.ds(0, _CHUNK)]], rbuf, gsem
        ).wait()

    def fire_write(j, rbuf, wsem):
        pltpu.make_async_copy(
            rbuf, out_hbm.at[pl.ds(base + j * _CHUNK, _CHUNK)], wsem
        ).start()

    def wait_write(j, rbuf, wsem):
        pltpu.make_async_copy(
            rbuf, out_hbm.at[pl.ds(base + j * _CHUNK, _CHUNK)], wsem
        ).wait()

    fire_gather(0, rb0, g0)
    fire_gather(1, rb1, g1)

    def body(i2, carry):
        for b, (rbuf, gsem, wsem) in enumerate(((rb0, g0, w0), (rb1, g1, w1))):
            j = i2 * 2 + b
            wait_gather(rbuf, gsem)
            fire_write(j, rbuf, wsem)
            wait_write(j, rbuf, wsem)

            @pl.when(j + 2 < _NCHUNK)
            def _():
                fire_gather(j + 2, rbuf, gsem)

        return carry

    lax.fori_loop(0, _NCHUNK // 2, body, 0)


def kernel(idx, table):
    table_cm = jnp.transpose(table)
    packed = _transpose_tc(table_cm)
    table_lin = jnp.reshape(packed, (_VP, _D))
    flat_idx = jnp.asarray(idx, jnp.int32).reshape(_N)
    out = _gather_kernel(flat_idx, table_lin)
    return out.reshape(_B, _T, _D)
